# baseline (speedup 1.0000x reference)
import functools

import jax
import jax.numpy as jnp
from jax import lax
from jax.experimental import pallas as pl
from jax.experimental.pallas import tpu as pltpu
from jax.experimental.pallas import tpu_sc as plsc


def _gather_rows_sc(emb_table_128, center_words, d_valid):
    B = center_words.shape[0]
    D = emb_table_128.shape[1]
    info = plsc.get_sparse_core_info()
    nc, ns = info.num_cores, info.num_subcores
    nw = nc * ns
    b_per_w = B // nw
    mesh = plsc.VectorSubcoreMesh(core_axis_name="c", subcore_axis_name="s")

    del d_valid

    @functools.partial(
        pl.kernel,
        mesh=mesh,
        out_type=jax.ShapeDtypeStruct((B, D), jnp.float32),
        scratch_types=[
            pltpu.VMEM((b_per_w,), jnp.int32),
            pltpu.VMEM((b_per_w, D), jnp.float32),
            pltpu.SemaphoreType.DMA,
        ],
    )
    def gather_kernel(idx_hbm, table_hbm, out_hbm, idx_v, rows_v, sem):
        wid = lax.axis_index("s") * nc + lax.axis_index("c")
        base = wid * b_per_w
        pltpu.sync_copy(idx_hbm.at[pl.ds(base, b_per_w)], idx_v)
        pltpu.async_copy(table_hbm.at[idx_v], rows_v, sem).wait()
        pltpu.sync_copy(rows_v, out_hbm.at[pl.ds(base, b_per_w)])

    return gather_kernel(center_words.astype(jnp.int32), emb_table_128)


def _decode_tc(e, dec_Wt, dec_b, v_tile=2048):
    B = e.shape[0]
    D, V = dec_Wt.shape
    grid = pl.cdiv(V, v_tile)
    b2 = dec_b.reshape(1, V)

    def body(e_ref, wt_ref, b_ref, out_ref):
        prod = lax.dot_general(
            wt_ref[...],
            e_ref[...][:, :D],
            dimension_numbers=(((0,), (1,)), ((), ())),
            preferred_element_type=jnp.float32,
        )
        ones = jnp.ones((1, B), dtype=jnp.float32)
        bias = lax.dot_general(
            b_ref[...],
            ones,
            dimension_numbers=(((0,), (0,)), ((), ())),
            preferred_element_type=jnp.float32,
        )
        out_ref[...] = prod + bias

    return pl.pallas_call(
        body,
        grid=(grid,),
        in_specs=[
            pl.BlockSpec((B, e.shape[1]), lambda i: (0, 0)),
            pl.BlockSpec((D, v_tile), lambda i: (0, i)),
            pl.BlockSpec((1, v_tile), lambda i: (0, i)),
        ],
        out_specs=pl.BlockSpec((v_tile, B), lambda i: (i, 0)),
        out_shape=jax.ShapeDtypeStruct((V, B), jnp.float32),
    )(e, dec_Wt, b2)


def _pad_transpose_tc(emb_t, c_tile=512):
    D, V = emb_t.shape

    def body(in_ref, out_ref):
        t = jnp.swapaxes(in_ref[...], 0, 1)
        out_ref[...] = jnp.concatenate([t, t], axis=1)

    return pl.pallas_call(
        body,
        grid=(pl.cdiv(V, c_tile),),
        in_specs=[pl.BlockSpec((D, c_tile), lambda i: (0, i))],
        out_specs=pl.BlockSpec((c_tile, 2 * D), lambda i: (i, 0)),
        out_shape=jax.ShapeDtypeStruct((V, 2 * D), jnp.float32),
    )(emb_t)


def kernel(center_words, emb_table, dec_W, dec_b):
    d = emb_table.shape[1]
    emb128 = _pad_transpose_tc(emb_table.T)
    e = _gather_rows_sc(emb128, center_words, d)
    out_t = _decode_tc(e, dec_W.T, dec_b)
    return out_t.T


# speedup vs baseline: 1.3642x; 1.3642x over previous
import functools

import jax
import jax.numpy as jnp
from jax import lax
from jax.experimental import pallas as pl
from jax.experimental.pallas import tpu as pltpu
from jax.experimental.pallas import tpu_sc as plsc


def _gather_rows_sc(emb_table_128, center_words, d_valid):
    B = center_words.shape[0]
    D = emb_table_128.shape[1]
    info = plsc.get_sparse_core_info()
    nc, ns = info.num_cores, info.num_subcores
    nw = nc * ns
    b_per_w = B // nw
    mesh = plsc.VectorSubcoreMesh(core_axis_name="c", subcore_axis_name="s")

    del d_valid

    @functools.partial(
        pl.kernel,
        mesh=mesh,
        out_type=jax.ShapeDtypeStruct((B, D), jnp.float32),
        scratch_types=[
            pltpu.VMEM((b_per_w,), jnp.int32),
            pltpu.VMEM((b_per_w, D), jnp.float32),
            pltpu.SemaphoreType.DMA,
        ],
    )
    def gather_kernel(idx_hbm, table_hbm, out_hbm, idx_v, rows_v, sem):
        wid = lax.axis_index("s") * nc + lax.axis_index("c")
        base = wid * b_per_w
        pltpu.sync_copy(idx_hbm.at[pl.ds(base, b_per_w)], idx_v)
        pltpu.async_copy(table_hbm.at[idx_v], rows_v, sem).wait()
        pltpu.sync_copy(rows_v, out_hbm.at[pl.ds(base, b_per_w)])

    return gather_kernel(center_words.astype(jnp.int32), emb_table_128)


def _decode_tc(e, dec_Wt, dec_b, v_tile=4096):
    B = e.shape[0]
    D, V = dec_Wt.shape
    grid = pl.cdiv(V, v_tile)
    b2 = dec_b.reshape(1, V)

    def body(e_ref, wt_ref, b_ref, out_ref):
        prod = lax.dot_general(
            wt_ref[...],
            e_ref[...][:, :D],
            dimension_numbers=(((0,), (1,)), ((), ())),
            preferred_element_type=jnp.float32,
        )
        ones = jnp.ones((1, B), dtype=jnp.float32)
        bias = lax.dot_general(
            b_ref[...],
            ones,
            dimension_numbers=(((0,), (0,)), ((), ())),
            preferred_element_type=jnp.float32,
        )
        out_ref[...] = prod + bias

    return pl.pallas_call(
        body,
        grid=(grid,),
        in_specs=[
            pl.BlockSpec((B, e.shape[1]), lambda i: (0, 0)),
            pl.BlockSpec((D, v_tile), lambda i: (0, i)),
            pl.BlockSpec((1, v_tile), lambda i: (0, i)),
        ],
        out_specs=pl.BlockSpec((v_tile, B), lambda i: (i, 0)),
        out_shape=jax.ShapeDtypeStruct((V, B), jnp.float32),
    )(e, dec_Wt, b2)


def kernel(center_words, emb_table, dec_W, dec_b):
    d = emb_table.shape[1]
    emb128 = jnp.pad(emb_table, ((0, 0), (0, 128 - d)))
    e = _gather_rows_sc(emb128, center_words, d)
    out_t = _decode_tc(e, dec_W.T, dec_b)
    return out_t.T


# speedup vs baseline: 1.3702x; 1.0044x over previous
import functools

import jax
import jax.numpy as jnp
from jax import lax
from jax.experimental import pallas as pl
from jax.experimental.pallas import tpu as pltpu
from jax.experimental.pallas import tpu_sc as plsc


def _gather_rows_sc(emb_table_128, center_words):
    B = center_words.shape[0]
    D = emb_table_128.shape[1]
    info = plsc.get_sparse_core_info()
    nc, ns = info.num_cores, info.num_subcores
    nw = nc * ns
    b_per_w = B // nw
    mesh = plsc.VectorSubcoreMesh(core_axis_name="c", subcore_axis_name="s")

    @functools.partial(
        pl.kernel,
        mesh=mesh,
        out_type=jax.ShapeDtypeStruct((B, D), jnp.float32),
        scratch_types=[
            pltpu.VMEM((b_per_w,), jnp.int32),
            pltpu.VMEM((b_per_w, D), jnp.float32),
            pltpu.SemaphoreType.DMA,
        ],
    )
    def gather_kernel(idx_hbm, table_hbm, out_hbm, idx_v, rows_v, sem):
        wid = lax.axis_index("s") * nc + lax.axis_index("c")
        base = wid * b_per_w
        pltpu.sync_copy(idx_hbm.at[pl.ds(base, b_per_w)], idx_v)
        pltpu.async_copy(table_hbm.at[idx_v], rows_v, sem).wait()
        pltpu.sync_copy(rows_v, out_hbm.at[pl.ds(base, b_per_w)])

    return gather_kernel(center_words.astype(jnp.int32), emb_table_128)


def _decode_tc(e, dec_Wt, dec_b, v_tile=4096):
    B = e.shape[0]
    D, V = dec_Wt.shape
    grid = pl.cdiv(V, v_tile)
    b2 = dec_b.reshape(1, V)

    def body(e_ref, wt_ref, b_ref, out_ref):
        prod = lax.dot_general(
            wt_ref[...],
            e_ref[...][:, :D],
            dimension_numbers=(((0,), (1,)), ((), ())),
            preferred_element_type=jnp.float32,
        )
        ones = jnp.ones((1, B), dtype=jnp.float32)
        bias = lax.dot_general(
            b_ref[...],
            ones,
            dimension_numbers=(((0,), (0,)), ((), ())),
            preferred_element_type=jnp.float32,
        )
        out_ref[...] = prod + bias

    return pl.pallas_call(
        body,
        grid=(grid,),
        in_specs=[
            pl.BlockSpec((B, e.shape[1]), lambda i: (0, 0)),
            pl.BlockSpec((D, v_tile), lambda i: (0, i)),
            pl.BlockSpec((1, v_tile), lambda i: (0, i)),
        ],
        out_specs=pl.BlockSpec((v_tile, B), lambda i: (i, 0)),
        out_shape=jax.ShapeDtypeStruct((V, B), jnp.float32),
    )(e, dec_Wt, b2)


def kernel(center_words, emb_table, dec_W, dec_b):
    d = emb_table.shape[1]
    emb128 = jnp.pad(emb_table, ((0, 0), (0, 128 - d)))
    e = _gather_rows_sc(emb128, center_words)
    out_t = _decode_tc(e, dec_W.T, dec_b)
    return out_t.T
